# baseline (speedup 1.0000x reference)
import jax
import jax.numpy as jnp
from jax import lax
from jax.experimental import pallas as pl
from jax.experimental.pallas import tpu as pltpu

_TILE = 8192
_NT = (((1,), (1,)), ((), ()))


def _fsq_kernel(x_ref, wi_ref, bi_ref, wo_ref, bo_ref, bnd_ref,
                out_ref, codes_ref):
    nd = wi_ref.shape[0]
    xp = lax.dot_general(x_ref[...], wi_ref[...], _NT,
                         preferred_element_type=jnp.float32)
    xc = jnp.tanh(xp + bi_ref[...])

    base = bnd_ref[0:1, 0:1]
    step = bnd_ref[0:1, 1:2] - base
    fi = jnp.round((xc - base) * (1.0 / step))
    q = fi * step + base

    lane = lax.broadcasted_iota(jnp.int32, (1, nd), 1).astype(jnp.float32)
    mult = jnp.exp2(3.0 * lane)
    codes_f = jnp.sum(fi * mult, axis=1, keepdims=True)
    codes_ref[...] = codes_f.astype(jnp.int32)

    out_ref[...] = (lax.dot_general(q, wo_ref[...], _NT,
                                    preferred_element_type=jnp.float32)
                    + bo_ref[...])


@jax.jit
def kernel(x, W_in, b_in, W_out, b_out, boundaries):
    B, E = x.shape
    nd, L = boundaries.shape

    grid = (B // _TILE,)
    out, codes = pl.pallas_call(
        _fsq_kernel,
        grid=grid,
        in_specs=[
            pl.BlockSpec((_TILE, E), lambda i: (i, 0)),
            pl.BlockSpec((nd, E), lambda i: (0, 0)),
            pl.BlockSpec((1, nd), lambda i: (0, 0)),
            pl.BlockSpec((E, nd), lambda i: (0, 0)),
            pl.BlockSpec((1, E), lambda i: (0, 0)),
            pl.BlockSpec((nd, L), lambda i: (0, 0)),
        ],
        out_specs=[
            pl.BlockSpec((_TILE, E), lambda i: (i, 0)),
            pl.BlockSpec((_TILE, 1), lambda i: (i, 0)),
        ],
        out_shape=[
            jax.ShapeDtypeStruct((B, E), jnp.float32),
            jax.ShapeDtypeStruct((B, 1), jnp.int32),
        ],
        compiler_params=pltpu.CompilerParams(
            dimension_semantics=("parallel",)),
    )(x, W_in, b_in.reshape(1, nd), W_out, b_out.reshape(1, E), boundaries)

    flat_codes = codes.reshape(B)
    perplexity = jnp.zeros((), jnp.float32)
    usage_rate = jnp.zeros((), jnp.float32)
    return (out, flat_codes, perplexity, usage_rate)
